# Initial kernel scaffold; baseline (speedup 1.0000x reference)
import jax
import jax.numpy as jnp
from jax.experimental import pallas as pl


def kernel(current_idx, target_idx, neighbor_indices, neighbor_mask, table, W1, b1, W2, b2, W3, b3, W4, b4):
    raise NotImplementedError("write your pallas kernel here")


# speedup vs baseline: 1.5531x; 1.5531x over previous
import functools

import jax
import jax.numpy as jnp
from jax import lax
from jax.experimental import pallas as pl
from jax.experimental.pallas import tpu as pltpu
from jax.experimental.pallas import tpu_sc as plsc

B = 4096
MAXN = 64
EMBED = 256
HIDDEN = 512

NEG_INF = float("-inf")

_NC = 2
_NS = 16
_NW = _NC * _NS
_NTOT = B + B + B * MAXN
_PER_W = _NTOT // _NW
_CHUNK = 128
_NCHUNK = _PER_W // _CHUNK


def _sc_gather(table, idx_all):
    mesh = plsc.VectorSubcoreMesh(core_axis_name="c", subcore_axis_name="s")

    @functools.partial(
        pl.kernel,
        mesh=mesh,
        out_type=jax.ShapeDtypeStruct((_NTOT, EMBED), table.dtype),
        scratch_types=[
            pltpu.VMEM((_PER_W,), jnp.int32),
            pltpu.VMEM((_CHUNK, EMBED), table.dtype),
            pltpu.SemaphoreType.DMA,
        ],
    )
    def gather_kernel(table_hbm, idx_hbm, out_hbm, idx_v, rows_v, sem):
        wid = lax.axis_index("s") * _NC + lax.axis_index("c")
        base = wid * _PER_W
        pltpu.sync_copy(idx_hbm.at[pl.ds(base, _PER_W)], idx_v)

        @pl.loop(0, _NCHUNK)
        def _(t):
            off = t * _CHUNK
            pltpu.async_copy(
                table_hbm.at[idx_v.at[pl.ds(off, _CHUNK)]], rows_v, sem
            ).wait()
            pltpu.sync_copy(rows_v, out_hbm.at[pl.ds(base + off, _CHUNK)])

    return gather_kernel(table, idx_all)


_R = 64


def _tc_body(cur_ref, tgt_ref, nbr_ref, mask_ref,
             w1c_ref, w1n_ref, b1_ref, w2_ref, b2_ref,
             w3c_ref, w3t_ref, w3n_ref, b3_ref, w4_ref, b4_ref,
             out_ref):
    f32 = jnp.float32
    bf16 = jnp.bfloat16

    mask3 = mask_ref[...]
    nbr = nbr_ref[...]
    nbr_b = nbr.astype(bf16)

    nbr3 = nbr.reshape(_R, MAXN, EMBED)
    masked_sum = jnp.sum(nbr3 * mask3, axis=1)
    cnt = jnp.maximum(jnp.sum(mask3[..., 0], axis=1, keepdims=True), 1.0)
    agg_b = (masked_sum / cnt).astype(bf16)

    cur_b = cur_ref[...].astype(bf16)
    tgt_b = tgt_ref[...].astype(bf16)

    h = jnp.dot(cur_b, w1c_ref[...], preferred_element_type=f32)
    h += jnp.dot(agg_b, w1n_ref[...], preferred_element_type=f32)
    h = jnp.maximum(h + b1_ref[...], 0.0)
    ctx = jnp.dot(h.astype(bf16), w2_ref[...], preferred_element_type=f32)
    ctx = ctx + b2_ref[...]

    a = jnp.dot(ctx.astype(bf16), w3c_ref[...], preferred_element_type=f32)
    a += jnp.dot(tgt_b, w3t_ref[...], preferred_element_type=f32)
    a = (a + b3_ref[...]).astype(bf16)

    n3 = jnp.dot(nbr_b, w3n_ref[...],
                 preferred_element_type=f32).astype(bf16)
    h2 = jnp.maximum(n3.reshape(_R, MAXN, HIDDEN) + a[:, None, :], 0.0)
    h2 = h2.reshape(_R * MAXN, HIDDEN)
    s = jnp.dot(h2, w4_ref[...], preferred_element_type=f32)
    s = s + b4_ref[...]

    mask_col = mask3.reshape(_R * MAXN, 1)
    out_ref[...] = jnp.where(mask_col > 0.0, s, NEG_INF)


def _tc_score(gathered, mask3, w1c, w1n, b1, w2, b2, w3c, w3t, w3n, b3, w4, b4):
    grid = (B // _R,)
    nbr_rows = _R * MAXN
    return pl.pallas_call(
        _tc_body,
        grid=grid,
        in_specs=[
            pl.BlockSpec((_R, EMBED), lambda i: (i, 0)),
            pl.BlockSpec((_R, EMBED), lambda i: (B // _R + i, 0)),
            pl.BlockSpec((nbr_rows, EMBED),
                         lambda i: (2 * B // nbr_rows + i, 0)),
            pl.BlockSpec((_R, MAXN, 1), lambda i: (i, 0, 0)),
            pl.BlockSpec((EMBED, HIDDEN), lambda i: (0, 0)),
            pl.BlockSpec((EMBED, HIDDEN), lambda i: (0, 0)),
            pl.BlockSpec((1, HIDDEN), lambda i: (0, 0)),
            pl.BlockSpec((HIDDEN, EMBED), lambda i: (0, 0)),
            pl.BlockSpec((1, EMBED), lambda i: (0, 0)),
            pl.BlockSpec((EMBED, HIDDEN), lambda i: (0, 0)),
            pl.BlockSpec((EMBED, HIDDEN), lambda i: (0, 0)),
            pl.BlockSpec((EMBED, HIDDEN), lambda i: (0, 0)),
            pl.BlockSpec((1, HIDDEN), lambda i: (0, 0)),
            pl.BlockSpec((HIDDEN, 1), lambda i: (0, 0)),
            pl.BlockSpec((1, 1), lambda i: (0, 0)),
        ],
        out_specs=pl.BlockSpec((nbr_rows, 1), lambda i: (i, 0)),
        out_shape=jax.ShapeDtypeStruct((B * MAXN, 1), jnp.float32),
    )(gathered, gathered, gathered, mask3,
      w1c, w1n, b1, w2, b2, w3c, w3t, w3n, b3, w4, b4)


def kernel(current_idx, target_idx, neighbor_indices, neighbor_mask,
           table, W1, b1, W2, b2, W3, b3, W4, b4):
    idx_all = jnp.concatenate(
        [current_idx, target_idx, neighbor_indices.reshape(-1)], axis=0)
    gathered = _sc_gather(table, idx_all)

    bf16 = jnp.bfloat16
    w1t = W1.T.astype(bf16)
    w1c, w1n = w1t[:EMBED], w1t[EMBED:]
    w3t_full = W3.T.astype(bf16)
    w3c, w3t, w3n = (w3t_full[:EMBED], w3t_full[EMBED:2 * EMBED],
                     w3t_full[2 * EMBED:])
    w2 = W2.T.astype(bf16)
    w4 = W4.T.astype(bf16)

    mask3 = neighbor_mask.astype(jnp.float32).reshape(B, MAXN, 1)
    scores = _tc_score(
        gathered, mask3,
        w1c, w1n, b1.reshape(1, HIDDEN),
        w2, b2.reshape(1, EMBED),
        w3c, w3t, w3n, b3.reshape(1, HIDDEN),
        w4, b4.reshape(1, 1))
    return scores.reshape(B, MAXN)
